# Initial kernel scaffold; baseline (speedup 1.0000x reference)
import jax
import jax.numpy as jnp
from jax import lax
from jax.experimental import pallas as pl
from jax.experimental.pallas import tpu as pltpu
from jax.experimental.pallas import tpu_sc as plsc

N = 10000
NPAD = 10240
E = 320000
D = 128
B = 64
NW = 32
CH = 128
NCHUNK = 80
EPAD = NW * NCHUNK * CH
ROWS_PER_SUB = NPAD // 16

_HIGH = jax.lax.Precision.HIGHEST


def _vector_mesh():
    return plsc.VectorSubcoreMesh(core_axis_name="c", subcore_axis_name="s")


def _deg_body(dst_hbm, out_hbm, dstv, ones_v, zbuf, acc):
    cid = lax.axis_index("c")
    sid = lax.axis_index("s")
    wid = cid * 16 + sid

    lanes = lax.iota(jnp.int32, 16)
    e1 = jnp.where(lanes == 0, 1.0, 0.0).astype(jnp.float32)
    zero = jnp.zeros_like(e1)

    @pl.loop(0, CH)
    def _(i):
        ones_v[i, :] = e1
        zbuf[i, :] = zero

    @pl.loop(0, ROWS_PER_SUB // CH)
    def _(k):
        pltpu.sync_copy(zbuf, acc.at[pl.ds(sid * ROWS_PER_SUB + k * CH, CH)])

    pltpu.sync_copy(dst_hbm.at[wid], dstv)
    plsc.subcore_barrier()

    @pl.loop(0, NCHUNK)
    def _(j):
        pltpu.sync_copy(ones_v, acc.at[dstv.at[j]], add=True)

    plsc.subcore_barrier()
    pltpu.sync_copy(
        acc.at[pl.ds(sid * ROWS_PER_SUB, ROWS_PER_SUB)],
        out_hbm.at[cid].at[pl.ds(sid * ROWS_PER_SUB, ROWS_PER_SUB)],
    )


def _deg_call(dst_p):
    k = pl.kernel(
        _deg_body,
        out_type=jax.ShapeDtypeStruct((2, NPAD, 16), jnp.float32),
        mesh=_vector_mesh(),
        scratch_types=[
            pltpu.VMEM((NCHUNK, CH), jnp.int32),
            pltpu.VMEM((CH, 16), jnp.float32),
            pltpu.VMEM((CH, 16), jnp.float32),
            pltpu.VMEM_SHARED((NPAD, 16), jnp.float32),
        ],
    )
    return k(dst_p)


def _agg_body(g_hbm, src_hbm, dst_hbm, out_hbm, srcv, dstv, bufa, bufb, acc,
              sema, semb):
    cid = lax.axis_index("c")
    sid = lax.axis_index("s")
    wid = cid * 16 + sid

    pltpu.sync_copy(
        g_hbm.at[pl.ds(sid * ROWS_PER_SUB, ROWS_PER_SUB)],
        acc.at[pl.ds(sid * ROWS_PER_SUB, ROWS_PER_SUB)],
    )
    pltpu.sync_copy(src_hbm.at[wid], srcv)
    pltpu.sync_copy(dst_hbm.at[wid], dstv)
    plsc.subcore_barrier()

    @pl.loop(0, NCHUNK, step=2)
    def _(j):
        ca = pltpu.async_copy(g_hbm.at[srcv.at[j]], bufa, sema)
        cb = pltpu.async_copy(g_hbm.at[srcv.at[j + 1]], bufb, semb)
        ca.wait()
        pltpu.sync_copy(bufa, acc.at[dstv.at[j]], add=True)
        cb.wait()
        pltpu.sync_copy(bufb, acc.at[dstv.at[j + 1]], add=True)

    plsc.subcore_barrier()
    pltpu.sync_copy(
        acc.at[pl.ds(sid * ROWS_PER_SUB, ROWS_PER_SUB)],
        out_hbm.at[cid].at[pl.ds(sid * ROWS_PER_SUB, ROWS_PER_SUB)],
    )


def _agg_call(g, src_p, dst_p):
    k = pl.kernel(
        _agg_body,
        out_type=jax.ShapeDtypeStruct((2, NPAD, D), jnp.float32),
        mesh=_vector_mesh(),
        scratch_types=[
            pltpu.VMEM((NCHUNK, CH), jnp.int32),
            pltpu.VMEM((NCHUNK, CH), jnp.int32),
            pltpu.VMEM((CH, D), jnp.float32),
            pltpu.VMEM((CH, D), jnp.float32),
            pltpu.VMEM_SHARED((NPAD, D), jnp.float32),
            pltpu.SemaphoreType.DMA,
            pltpu.SemaphoreType.DMA,
        ],
    )
    return k(g, src_p, dst_p)


def _dinv_from(degp):
    deg = degp[0, :, 0:1] + degp[1, :, 0:1] + 1.0
    return lax.rsqrt(deg)


def _tc1_body(xp_ref, w1_ref, degp_ref, g1_ref):
    dinv = _dinv_from(degp_ref[...])
    h = jnp.dot(xp_ref[...], w1_ref[...], precision=_HIGH,
                preferred_element_type=jnp.float32)
    g1_ref[...] = h * dinv


def _tc2_body(degp_ref, p_ref, g_ref, b_ref, w_ref, out_ref):
    dinv = _dinv_from(degp_ref[...])
    u = p_ref[0] + p_ref[1] - g_ref[...]
    z = jnp.maximum(u * dinv + b_ref[...], 0.0)
    out_ref[...] = jnp.dot(z, w_ref[...], precision=_HIGH,
                           preferred_element_type=jnp.float32) * dinv


def _tc3_body(degp_ref, p_ref, g_ref, b_ref, batch_ref, wl_ref, bl_ref,
              out_ref):
    dinv = _dinv_from(degp_ref[...])
    u = p_ref[0] + p_ref[1] - g_ref[...]
    z = jnp.maximum(u * dinv + b_ref[...], 0.0)
    gid = lax.broadcasted_iota(jnp.int32, (1, B), 1)
    m = (batch_ref[...] == gid).astype(jnp.float32)
    sums = lax.dot_general(m, z, (((0,), (0,)), ((), ())), precision=_HIGH,
                           preferred_element_type=jnp.float32)
    counts = jnp.sum(m, axis=0)[:, None]
    pooled = sums / jnp.maximum(counts, 1.0)
    out_ref[...] = jnp.dot(pooled, wl_ref[...], precision=_HIGH,
                           preferred_element_type=jnp.float32) + bl_ref[...]


def _tc_call(body, out_shape, *args):
    return pl.pallas_call(body, out_shape=out_shape)(*args)


def kernel(x, ei, batch, W1, b1, W2, b2, Wl, bl):
    pad = EPAD - E
    idx = jnp.arange(pad, dtype=jnp.int32)
    src_p = jnp.concatenate([ei[0], (idx * 37) % N]).reshape(NW, NCHUNK, CH)
    dst_p = jnp.concatenate([ei[1], N + idx % (NPAD - N)]).reshape(
        NW, NCHUNK, CH)
    x_p = jnp.pad(x, ((0, NPAD - N), (0, 0)))
    batch_p = jnp.pad(batch, (0, NPAD - N), constant_values=B).reshape(
        NPAD, 1)

    degp = _deg_call(dst_p)

    g1 = _tc_call(_tc1_body, jax.ShapeDtypeStruct((NPAD, D), jnp.float32),
                  x_p, W1, degp)
    p1 = _agg_call(g1, src_p, dst_p)
    g2 = _tc_call(_tc2_body, jax.ShapeDtypeStruct((NPAD, D), jnp.float32),
                  degp, p1, g1, b1.reshape(1, D), W2)
    p2 = _agg_call(g2, src_p, dst_p)
    out = _tc_call(_tc3_body, jax.ShapeDtypeStruct((B, D), jnp.float32),
                   degp, p2, g2, b2.reshape(1, D), batch_p, Wl,
                   bl.reshape(1, D))
    return out


# speedup vs baseline: 21.2167x; 21.2167x over previous
import jax
import jax.numpy as jnp
from jax import lax
from jax.experimental import pallas as pl
from jax.experimental.pallas import tpu as pltpu
from jax.experimental.pallas import tpu_sc as plsc

N = 10000
NPAD = 10240
E = 320000
D = 128
B = 64
NW = 32
CH = 128
NCHUNK = 80
NCHUNK2 = 160
EPAD = NW * NCHUNK * CH
HD = D // 2
ROWS_PER_SUB = NPAD // 16

_HIGH = jax.lax.Precision.HIGHEST


def _vector_mesh():
    return plsc.VectorSubcoreMesh(core_axis_name="c", subcore_axis_name="s")


_SC_PARAMS = pltpu.CompilerParams(use_tc_tiling_on_sc=False)


def _deg_body(dst_hbm, out_hbm, dstv, ones_v, zbuf, acc):
    cid = lax.axis_index("c")
    sid = lax.axis_index("s")
    wid = cid * 16 + sid

    lanes = lax.iota(jnp.int32, 16)
    e1 = jnp.where(lanes == 0, 1.0, 0.0).astype(jnp.float32)
    zero = jnp.zeros_like(e1)

    @pl.loop(0, CH)
    def _(i):
        ones_v[i, :] = e1
        zbuf[i, :] = zero

    @pl.loop(0, ROWS_PER_SUB // CH)
    def _(k):
        pltpu.sync_copy(zbuf, acc.at[pl.ds(sid * ROWS_PER_SUB + k * CH, CH)])

    pltpu.sync_copy(dst_hbm.at[wid], dstv)
    plsc.subcore_barrier()

    @pl.loop(0, NCHUNK)
    def _(j):
        pltpu.sync_copy(ones_v, acc.at[dstv.at[j]], add=True)

    plsc.subcore_barrier()
    pltpu.sync_copy(
        acc.at[pl.ds(sid * ROWS_PER_SUB, ROWS_PER_SUB)],
        out_hbm.at[cid].at[pl.ds(sid * ROWS_PER_SUB, ROWS_PER_SUB)],
    )


def _deg_call(dst_p):
    k = pl.kernel(
        _deg_body,
        out_type=jax.ShapeDtypeStruct((2, NPAD, 16), jnp.float32),
        mesh=_vector_mesh(),
        compiler_params=_SC_PARAMS,
        scratch_types=[
            pltpu.VMEM((NCHUNK, CH), jnp.int32),
            pltpu.VMEM((CH, 16), jnp.float32),
            pltpu.VMEM((CH, 16), jnp.float32),
            pltpu.VMEM_SHARED((NPAD, 16), jnp.float32),
        ],
    )
    return k(dst_p)


def _agg_body(g_hbm, src_hbm, dst_hbm, out_hbm, srcv, dstv, bufa, bufb, acc,
              sema, semb):
    cid = lax.axis_index("c")
    sid = lax.axis_index("s")
    gh = g_hbm.at[cid]

    pltpu.sync_copy(
        gh.at[pl.ds(sid * ROWS_PER_SUB, ROWS_PER_SUB)],
        acc.at[pl.ds(sid * ROWS_PER_SUB, ROWS_PER_SUB)],
    )
    pltpu.sync_copy(src_hbm.at[sid], srcv)
    pltpu.sync_copy(dst_hbm.at[sid], dstv)
    plsc.subcore_barrier()

    @pl.loop(0, NCHUNK2, step=2)
    def _(j):
        ca = pltpu.async_copy(gh.at[srcv.at[j]], bufa, sema)
        cb = pltpu.async_copy(gh.at[srcv.at[j + 1]], bufb, semb)
        ca.wait()
        pltpu.sync_copy(bufa, acc.at[dstv.at[j]], add=True)
        cb.wait()
        pltpu.sync_copy(bufb, acc.at[dstv.at[j + 1]], add=True)

    plsc.subcore_barrier()
    pltpu.sync_copy(
        acc.at[pl.ds(sid * ROWS_PER_SUB, ROWS_PER_SUB)],
        out_hbm.at[cid].at[pl.ds(sid * ROWS_PER_SUB, ROWS_PER_SUB)],
    )


def _agg_call(g2h, src_p, dst_p):
    k = pl.kernel(
        _agg_body,
        out_type=jax.ShapeDtypeStruct((2, NPAD, HD), jnp.float32),
        mesh=_vector_mesh(),
        compiler_params=_SC_PARAMS,
        scratch_types=[
            pltpu.VMEM((NCHUNK2, CH), jnp.int32),
            pltpu.VMEM((NCHUNK2, CH), jnp.int32),
            pltpu.VMEM((CH, HD), jnp.float32),
            pltpu.VMEM((CH, HD), jnp.float32),
            pltpu.VMEM_SHARED((NPAD, HD), jnp.float32),
            pltpu.SemaphoreType.DMA,
            pltpu.SemaphoreType.DMA,
        ],
    )
    return k(g2h, src_p, dst_p)


def _dinv_from(degp):
    deg = degp[0, :, 0:1] + degp[1, :, 0:1] + 1.0
    return lax.rsqrt(deg)


def _store_halves(out_ref, y):
    out_ref[0] = y[:, :HD]
    out_ref[1] = y[:, HD:]


def _tc1_body(xp_ref, w1_ref, degp_ref, g1_ref):
    dinv = _dinv_from(degp_ref[...])
    h = jnp.dot(xp_ref[...], w1_ref[...], precision=_HIGH,
                preferred_element_type=jnp.float32)
    _store_halves(g1_ref, h * dinv)


def _tc2_body(degp_ref, p_ref, b_ref, w_ref, out_ref):
    dinv = _dinv_from(degp_ref[...])
    u = jnp.concatenate([p_ref[0], p_ref[1]], axis=1)
    z = jnp.maximum(u * dinv + b_ref[...], 0.0)
    _store_halves(out_ref, jnp.dot(z, w_ref[...], precision=_HIGH,
                                   preferred_element_type=jnp.float32) * dinv)


def _tc3_body(degp_ref, p_ref, b_ref, batch_ref, wl_ref, bl_ref,
              out_ref):
    dinv = _dinv_from(degp_ref[...])
    u = jnp.concatenate([p_ref[0], p_ref[1]], axis=1)
    z = jnp.maximum(u * dinv + b_ref[...], 0.0)
    gid = lax.broadcasted_iota(jnp.int32, (1, B), 1)
    m = (batch_ref[...] == gid).astype(jnp.float32)
    sums = lax.dot_general(m, z, (((0,), (0,)), ((), ())), precision=_HIGH,
                           preferred_element_type=jnp.float32)
    counts = jnp.sum(m, axis=0)[:, None]
    pooled = sums / jnp.maximum(counts, 1.0)
    out_ref[...] = jnp.dot(pooled, wl_ref[...], precision=_HIGH,
                           preferred_element_type=jnp.float32) + bl_ref[...]


def _tc_call(body, out_shape, *args):
    return pl.pallas_call(body, out_shape=out_shape)(*args)


def kernel(x, ei, batch, W1, b1, W2, b2, Wl, bl):
    pad = EPAD - E
    idx = jnp.arange(pad, dtype=jnp.int32)
    src_flat = jnp.concatenate([ei[0], (idx * 37) % N])
    dst_flat = jnp.concatenate([ei[1], N + idx % (NPAD - N)])
    src_p = src_flat.reshape(16, NCHUNK2, CH)
    dst_p = dst_flat.reshape(16, NCHUNK2, CH)
    dst_p32 = dst_flat.reshape(NW, NCHUNK, CH)
    x_p = jnp.pad(x, ((0, NPAD - N), (0, 0)))
    batch_p = jnp.pad(batch, (0, NPAD - N), constant_values=B).reshape(
        NPAD, 1)

    degp = _deg_call(dst_p32)

    g1 = _tc_call(_tc1_body, jax.ShapeDtypeStruct((2, NPAD, HD), jnp.float32),
                  x_p, W1, degp)
    p1 = _agg_call(g1, src_p, dst_p)
    g2 = _tc_call(_tc2_body, jax.ShapeDtypeStruct((2, NPAD, HD), jnp.float32),
                  degp, p1, b1.reshape(1, D), W2)
    p2 = _agg_call(g2, src_p, dst_p)
    out = _tc_call(_tc3_body, jax.ShapeDtypeStruct((B, D), jnp.float32),
                   degp, p2, b2.reshape(1, D), batch_p, Wl,
                   bl.reshape(1, D))
    return out


# speedup vs baseline: 28.2747x; 1.3327x over previous
import jax
import jax.numpy as jnp
from jax import lax
from jax.experimental import pallas as pl
from jax.experimental.pallas import tpu as pltpu
from jax.experimental.pallas import tpu_sc as plsc

N = 10000
NPAD = 10240
E = 320000
D = 128
B = 64
NW = 32
CH = 128
NCHUNK = 80
NCHUNK2 = 160
EPAD = NW * NCHUNK * CH
HD = D // 2
ROWS_PER_SUB = NPAD // 16

_HIGH = jax.lax.Precision.HIGHEST


def _vector_mesh():
    return plsc.VectorSubcoreMesh(core_axis_name="c", subcore_axis_name="s")


_SC_PARAMS = pltpu.CompilerParams(use_tc_tiling_on_sc=False)


def _deg_body(dst_hbm, out_hbm, dstv, ones_v, zbuf, acc):
    cid = lax.axis_index("c")
    sid = lax.axis_index("s")
    wid = cid * 16 + sid

    lanes = lax.iota(jnp.int32, 16)
    e1 = jnp.where(lanes == 0, 1.0, 0.0).astype(jnp.float32)
    zero = jnp.zeros_like(e1)

    @pl.loop(0, CH)
    def _(i):
        ones_v[i, :] = e1
        zbuf[i, :] = zero

    @pl.loop(0, ROWS_PER_SUB // CH)
    def _(k):
        pltpu.sync_copy(zbuf, acc.at[pl.ds(sid * ROWS_PER_SUB + k * CH, CH)])

    pltpu.sync_copy(dst_hbm.at[wid], dstv)
    plsc.subcore_barrier()

    @pl.loop(0, NCHUNK)
    def _(j):
        pltpu.sync_copy(ones_v, acc.at[dstv.at[j]], add=True)

    plsc.subcore_barrier()
    pltpu.sync_copy(
        acc.at[pl.ds(sid * ROWS_PER_SUB, ROWS_PER_SUB)],
        out_hbm.at[cid].at[pl.ds(sid * ROWS_PER_SUB, ROWS_PER_SUB)],
    )


def _deg_call(dst_p):
    k = pl.kernel(
        _deg_body,
        out_type=jax.ShapeDtypeStruct((2, NPAD, 16), jnp.float32),
        mesh=_vector_mesh(),
        compiler_params=_SC_PARAMS,
        scratch_types=[
            pltpu.VMEM((NCHUNK, CH), jnp.int32),
            pltpu.VMEM((CH, 16), jnp.float32),
            pltpu.VMEM((CH, 16), jnp.float32),
            pltpu.VMEM_SHARED((NPAD, 16), jnp.float32),
        ],
    )
    return k(dst_p)


NBUF = 4


def _agg_body(g_hbm, src_hbm, dst_hbm, out_hbm, srcv, dstv, b0, b1, b2, b3,
              acc, g0, g1, g2, g3, s0, s1, s2, s3):
    cid = lax.axis_index("c")
    sid = lax.axis_index("s")
    gh = g_hbm.at[cid]
    bufs = (b0, b1, b2, b3)
    gsem = (g0, g1, g2, g3)
    ssem = (s0, s1, s2, s3)

    pltpu.sync_copy(
        gh.at[pl.ds(sid * ROWS_PER_SUB, ROWS_PER_SUB)],
        acc.at[pl.ds(sid * ROWS_PER_SUB, ROWS_PER_SUB)],
    )
    pltpu.sync_copy(src_hbm.at[sid], srcv)
    pltpu.sync_copy(dst_hbm.at[sid], dstv)
    plsc.subcore_barrier()

    for i in range(NBUF):
        pltpu.async_copy(gh.at[srcv.at[i]], bufs[i], gsem[i])

    @pl.loop(0, NCHUNK2 // NBUF - 1)
    def _(t):
        c0 = t * NBUF
        for i in range(NBUF):
            pltpu.make_async_copy(gh.at[srcv.at[c0 + i]], bufs[i],
                                  gsem[i]).wait()
            pltpu.async_copy(bufs[i], acc.at[dstv.at[c0 + i]], ssem[i],
                             add=True)
        for i in range(NBUF):
            pltpu.make_async_copy(bufs[i], acc.at[dstv.at[c0 + i]],
                                  ssem[i]).wait()
            pltpu.async_copy(gh.at[srcv.at[c0 + NBUF + i]], bufs[i], gsem[i])

    cl = NCHUNK2 - NBUF
    for i in range(NBUF):
        pltpu.make_async_copy(gh.at[srcv.at[cl + i]], bufs[i], gsem[i]).wait()
        pltpu.async_copy(bufs[i], acc.at[dstv.at[cl + i]], ssem[i], add=True)
    for i in range(NBUF):
        pltpu.make_async_copy(bufs[i], acc.at[dstv.at[cl + i]],
                              ssem[i]).wait()

    plsc.subcore_barrier()
    pltpu.sync_copy(
        acc.at[pl.ds(sid * ROWS_PER_SUB, ROWS_PER_SUB)],
        out_hbm.at[cid].at[pl.ds(sid * ROWS_PER_SUB, ROWS_PER_SUB)],
    )


def _agg_call(g2h, src_p, dst_p):
    k = pl.kernel(
        _agg_body,
        out_type=jax.ShapeDtypeStruct((2, NPAD, HD), jnp.float32),
        mesh=_vector_mesh(),
        compiler_params=_SC_PARAMS,
        scratch_types=[
            pltpu.VMEM((NCHUNK2, CH), jnp.int32),
            pltpu.VMEM((NCHUNK2, CH), jnp.int32),
        ] + [pltpu.VMEM((CH, HD), jnp.float32)] * NBUF + [
            pltpu.VMEM_SHARED((NPAD, HD), jnp.float32),
        ] + [pltpu.SemaphoreType.DMA] * (2 * NBUF),
    )
    return k(g2h, src_p, dst_p)


def _dinv_from(degp):
    deg = degp[0, :, 0:1] + degp[1, :, 0:1] + 1.0
    return lax.rsqrt(deg)


def _store_halves(out_ref, y):
    out_ref[0] = y[:, :HD]
    out_ref[1] = y[:, HD:]


def _tc0_body(xp_ref, w1_ref, h_ref):
    h_ref[...] = jnp.dot(xp_ref[...], w1_ref[...], precision=_HIGH,
                         preferred_element_type=jnp.float32)


def _tc1_body(h_ref, degp_ref, g1_ref):
    dinv = _dinv_from(degp_ref[...])
    _store_halves(g1_ref, h_ref[...] * dinv)


def _tc2_body(degp_ref, p_ref, b_ref, w_ref, out_ref):
    dinv = _dinv_from(degp_ref[...])
    u = jnp.concatenate([p_ref[0], p_ref[1]], axis=1)
    z = jnp.maximum(u * dinv + b_ref[...], 0.0)
    _store_halves(out_ref, jnp.dot(z, w_ref[...], precision=_HIGH,
                                   preferred_element_type=jnp.float32) * dinv)


def _tc3_body(degp_ref, p_ref, b_ref, batch_ref, wl_ref, bl_ref,
              out_ref):
    dinv = _dinv_from(degp_ref[...])
    u = jnp.concatenate([p_ref[0], p_ref[1]], axis=1)
    z = jnp.maximum(u * dinv + b_ref[...], 0.0)
    gid = lax.broadcasted_iota(jnp.int32, (1, B), 1)
    m = (batch_ref[...] == gid).astype(jnp.float32)
    sums = lax.dot_general(m, z, (((0,), (0,)), ((), ())), precision=_HIGH,
                           preferred_element_type=jnp.float32)
    counts = jnp.sum(m, axis=0)[:, None]
    pooled = sums / jnp.maximum(counts, 1.0)
    out_ref[...] = jnp.dot(pooled, wl_ref[...], precision=_HIGH,
                           preferred_element_type=jnp.float32) + bl_ref[...]


def _tc_call(body, out_shape, *args):
    return pl.pallas_call(body, out_shape=out_shape)(*args)


def kernel(x, ei, batch, W1, b1, W2, b2, Wl, bl):
    pad = EPAD - E
    idx = jnp.arange(pad, dtype=jnp.int32)
    src_flat = jnp.concatenate([ei[0], (idx * 37) % N])
    dst_flat = jnp.concatenate([ei[1], N + idx % (NPAD - N)])
    src_p = src_flat.reshape(16, NCHUNK2, CH)
    dst_p = dst_flat.reshape(16, NCHUNK2, CH)
    dst_p32 = dst_flat.reshape(NW, NCHUNK, CH)
    x_p = jnp.pad(x, ((0, NPAD - N), (0, 0)))
    batch_p = jnp.pad(batch, (0, NPAD - N), constant_values=B).reshape(
        NPAD, 1)

    degp = _deg_call(dst_p32)

    h1 = _tc_call(_tc0_body, jax.ShapeDtypeStruct((NPAD, D), jnp.float32),
                  x_p, W1)
    g1 = _tc_call(_tc1_body, jax.ShapeDtypeStruct((2, NPAD, HD), jnp.float32),
                  h1, degp)
    p1 = _agg_call(g1, src_p, dst_p)
    g2 = _tc_call(_tc2_body, jax.ShapeDtypeStruct((2, NPAD, HD), jnp.float32),
                  degp, p1, b1.reshape(1, D), W2)
    p2 = _agg_call(g2, src_p, dst_p)
    out = _tc_call(_tc3_body, jax.ShapeDtypeStruct((B, D), jnp.float32),
                   degp, p2, b2.reshape(1, D), batch_p, Wl,
                   bl.reshape(1, D))
    return out


# speedup vs baseline: 28.6432x; 1.0130x over previous
import jax
import jax.numpy as jnp
from jax import lax
from jax.experimental import pallas as pl
from jax.experimental.pallas import tpu as pltpu
from jax.experimental.pallas import tpu_sc as plsc

N = 10000
NPAD = 10240
E = 320000
D = 128
B = 64
NW = 32
CH = 128
NCHUNK = 80
NCHUNK2 = 160
EPAD = NW * NCHUNK * CH
HD = D // 2
ROWS_PER_SUB = NPAD // 16

_HIGH = jax.lax.Precision.HIGHEST


def _vector_mesh():
    return plsc.VectorSubcoreMesh(core_axis_name="c", subcore_axis_name="s")


_SC_PARAMS = pltpu.CompilerParams(use_tc_tiling_on_sc=False)


def _deg_body(dst_hbm, out_hbm, dstv, ones_v, zbuf, acc):
    cid = lax.axis_index("c")
    sid = lax.axis_index("s")
    wid = cid * 16 + sid

    lanes = lax.iota(jnp.int32, 16)
    e1 = jnp.where(lanes == 0, 1.0, 0.0).astype(jnp.float32)
    zero = jnp.zeros_like(e1)

    @pl.loop(0, CH)
    def _(i):
        ones_v[i, :] = e1
        zbuf[i, :] = zero

    @pl.loop(0, ROWS_PER_SUB // CH)
    def _(k):
        pltpu.sync_copy(zbuf, acc.at[pl.ds(sid * ROWS_PER_SUB + k * CH, CH)])

    pltpu.sync_copy(dst_hbm.at[wid], dstv)
    plsc.subcore_barrier()

    @pl.loop(0, NCHUNK)
    def _(j):
        pltpu.sync_copy(ones_v, acc.at[dstv.at[j]], add=True)

    plsc.subcore_barrier()
    pltpu.sync_copy(
        acc.at[pl.ds(sid * ROWS_PER_SUB, ROWS_PER_SUB)],
        out_hbm.at[cid].at[pl.ds(sid * ROWS_PER_SUB, ROWS_PER_SUB)],
    )


def _deg_call(dst_p):
    k = pl.kernel(
        _deg_body,
        out_type=jax.ShapeDtypeStruct((2, NPAD, 16), jnp.float32),
        mesh=_vector_mesh(),
        compiler_params=_SC_PARAMS,
        scratch_types=[
            pltpu.VMEM((NCHUNK, CH), jnp.int32),
            pltpu.VMEM((CH, 16), jnp.float32),
            pltpu.VMEM((CH, 16), jnp.float32),
            pltpu.VMEM_SHARED((NPAD, 16), jnp.float32),
        ],
    )
    return k(dst_p)


NBUF = 5


def _agg_body(g_hbm, src_hbm, dst_hbm, out_hbm, srcv, dstv, *bufs_and_sems):
    cid = lax.axis_index("c")
    sid = lax.axis_index("s")
    gh = g_hbm.at[cid]
    bufs = bufs_and_sems[:NBUF]
    acc = bufs_and_sems[NBUF]
    gsem = bufs_and_sems[NBUF + 1:2 * NBUF + 1]
    ssem = bufs_and_sems[2 * NBUF + 1:]

    pltpu.sync_copy(
        gh.at[pl.ds(sid * ROWS_PER_SUB, ROWS_PER_SUB)],
        acc.at[pl.ds(sid * ROWS_PER_SUB, ROWS_PER_SUB)],
    )
    pltpu.sync_copy(src_hbm.at[sid], srcv)
    pltpu.sync_copy(dst_hbm.at[sid], dstv)
    plsc.subcore_barrier()

    for i in range(NBUF):
        pltpu.async_copy(gh.at[srcv.at[i]], bufs[i], gsem[i])

    @pl.loop(0, NCHUNK2 // NBUF - 1)
    def _(t):
        c0 = t * NBUF
        for i in range(NBUF):
            pltpu.make_async_copy(gh.at[srcv.at[c0 + i]], bufs[i],
                                  gsem[i]).wait()
            pltpu.async_copy(bufs[i], acc.at[dstv.at[c0 + i]], ssem[i],
                             add=True)
        for i in range(NBUF):
            pltpu.make_async_copy(bufs[i], acc.at[dstv.at[c0 + i]],
                                  ssem[i]).wait()
            pltpu.async_copy(gh.at[srcv.at[c0 + NBUF + i]], bufs[i], gsem[i])

    cl = NCHUNK2 - NBUF
    for i in range(NBUF):
        pltpu.make_async_copy(gh.at[srcv.at[cl + i]], bufs[i], gsem[i]).wait()
        pltpu.async_copy(bufs[i], acc.at[dstv.at[cl + i]], ssem[i], add=True)
    for i in range(NBUF):
        pltpu.make_async_copy(bufs[i], acc.at[dstv.at[cl + i]],
                              ssem[i]).wait()

    plsc.subcore_barrier()
    pltpu.sync_copy(
        acc.at[pl.ds(sid * ROWS_PER_SUB, ROWS_PER_SUB)],
        out_hbm.at[cid].at[pl.ds(sid * ROWS_PER_SUB, ROWS_PER_SUB)],
    )


def _agg_call(g2h, src_p, dst_p):
    k = pl.kernel(
        _agg_body,
        out_type=jax.ShapeDtypeStruct((2, NPAD, HD), jnp.float32),
        mesh=_vector_mesh(),
        compiler_params=_SC_PARAMS,
        scratch_types=[
            pltpu.VMEM((NCHUNK2, CH), jnp.int32),
            pltpu.VMEM((NCHUNK2, CH), jnp.int32),
        ] + [pltpu.VMEM((CH, HD), jnp.float32)] * NBUF + [
            pltpu.VMEM_SHARED((NPAD, HD), jnp.float32),
        ] + [pltpu.SemaphoreType.DMA] * (2 * NBUF),
    )
    return k(g2h, src_p, dst_p)


def _dinv_from(degp):
    deg = degp[0, :, 0:1] + degp[1, :, 0:1] + 1.0
    return lax.rsqrt(deg)


def _store_halves(out_ref, y):
    out_ref[0] = y[:, :HD]
    out_ref[1] = y[:, HD:]


def _tc0_body(xp_ref, w1_ref, h_ref):
    h_ref[...] = jnp.dot(xp_ref[...], w1_ref[...], precision=_HIGH,
                         preferred_element_type=jnp.float32)


def _tc1_body(h_ref, degp_ref, g1_ref):
    dinv = _dinv_from(degp_ref[...])
    _store_halves(g1_ref, h_ref[...] * dinv)


def _tc2_body(degp_ref, p_ref, b_ref, w_ref, out_ref):
    dinv = _dinv_from(degp_ref[...])
    u = jnp.concatenate([p_ref[0], p_ref[1]], axis=1)
    z = jnp.maximum(u * dinv + b_ref[...], 0.0)
    _store_halves(out_ref, jnp.dot(z, w_ref[...], precision=_HIGH,
                                   preferred_element_type=jnp.float32) * dinv)


def _tc3_body(degp_ref, p_ref, b_ref, batch_ref, wl_ref, bl_ref,
              out_ref):
    dinv = _dinv_from(degp_ref[...])
    u = jnp.concatenate([p_ref[0], p_ref[1]], axis=1)
    z = jnp.maximum(u * dinv + b_ref[...], 0.0)
    gid = lax.broadcasted_iota(jnp.int32, (1, B), 1)
    m = (batch_ref[...] == gid).astype(jnp.float32)
    sums = lax.dot_general(m, z, (((0,), (0,)), ((), ())), precision=_HIGH,
                           preferred_element_type=jnp.float32)
    counts = jnp.sum(m, axis=0)[:, None]
    pooled = sums / jnp.maximum(counts, 1.0)
    out_ref[...] = jnp.dot(pooled, wl_ref[...], precision=_HIGH,
                           preferred_element_type=jnp.float32) + bl_ref[...]


def _tc_call(body, out_shape, *args):
    return pl.pallas_call(body, out_shape=out_shape)(*args)


def kernel(x, ei, batch, W1, b1, W2, b2, Wl, bl):
    pad = EPAD - E
    idx = jnp.arange(pad, dtype=jnp.int32)
    src_flat = jnp.concatenate([ei[0], (idx * 37) % N])
    dst_flat = jnp.concatenate([ei[1], N + idx % (NPAD - N)])
    src_p = src_flat.reshape(16, NCHUNK2, CH)
    dst_p = dst_flat.reshape(16, NCHUNK2, CH)
    dst_p32 = dst_flat.reshape(NW, NCHUNK, CH)
    x_p = jnp.pad(x, ((0, NPAD - N), (0, 0)))
    batch_p = jnp.pad(batch, (0, NPAD - N), constant_values=B).reshape(
        NPAD, 1)

    degp = _deg_call(dst_p32)

    h1 = _tc_call(_tc0_body, jax.ShapeDtypeStruct((NPAD, D), jnp.float32),
                  x_p, W1)
    g1 = _tc_call(_tc1_body, jax.ShapeDtypeStruct((2, NPAD, HD), jnp.float32),
                  h1, degp)
    p1 = _agg_call(g1, src_p, dst_p)
    g2 = _tc_call(_tc2_body, jax.ShapeDtypeStruct((2, NPAD, HD), jnp.float32),
                  degp, p1, b1.reshape(1, D), W2)
    p2 = _agg_call(g2, src_p, dst_p)
    out = _tc_call(_tc3_body, jax.ShapeDtypeStruct((B, D), jnp.float32),
                   degp, p2, b2.reshape(1, D), batch_p, Wl,
                   bl.reshape(1, D))
    return out


# speedup vs baseline: 29.4278x; 1.0274x over previous
import jax
import jax.numpy as jnp
from jax import lax
from jax.experimental import pallas as pl
from jax.experimental.pallas import tpu as pltpu
from jax.experimental.pallas import tpu_sc as plsc

N = 10000
NPAD = 10240
E = 320000
D = 128
B = 64
NW = 32
CH = 128
NCHUNK = 80
NCHUNK2 = 160
EPAD = NW * NCHUNK * CH
HD = D // 2
ROWS_PER_SUB = NPAD // 16

_HIGH = jax.lax.Precision.HIGHEST


def _vector_mesh():
    return plsc.VectorSubcoreMesh(core_axis_name="c", subcore_axis_name="s")


_SC_PARAMS = pltpu.CompilerParams(use_tc_tiling_on_sc=False)
_SC_PARAMS_NL = pltpu.CompilerParams(use_tc_tiling_on_sc=False,
                                     needs_layout_passes=False)


CHD = 80
NCHD = 125


def _deg_body(dst_hbm, out_hbm, dstv, hist):
    cid = lax.axis_index("c")
    sid = lax.axis_index("s")
    wid = cid * 16 + sid

    zero = jnp.zeros((16,), jnp.float32)
    ones = jnp.ones((16,), jnp.float32)

    @pl.loop(0, D // 16)
    def _(k):
        @pl.loop(0, CHD)
        def _(i):
            hist[i, pl.ds(k * 16, 16)] = zero

    pltpu.sync_copy(dst_hbm.at[wid], dstv)

    @pl.loop(0, NCHD)
    def _(j):
        for k in range(CHD // 16):
            v = dstv[j, pl.ds(k * 16, 16)]
            plsc.addupdate_scatter(hist, [v >> 7, v & 127], ones)

    pltpu.sync_copy(hist, out_hbm.at[wid])


def _deg_call(dst_deg):
    k = pl.kernel(
        _deg_body,
        out_type=jax.ShapeDtypeStruct((NW, CHD, D), jnp.float32),
        mesh=_vector_mesh(),
        compiler_params=_SC_PARAMS_NL,
        scratch_types=[
            pltpu.VMEM((NCHD, CHD), jnp.int32),
            pltpu.VMEM((CHD, D), jnp.float32),
        ],
    )
    return k(dst_deg)


NBUF = 5


def _agg_body(g_hbm, src_hbm, dst_hbm, out_hbm, srcv, dstv, *bufs_and_sems):
    cid = lax.axis_index("c")
    sid = lax.axis_index("s")
    gh = g_hbm.at[cid]
    bufs = bufs_and_sems[:NBUF]
    acc = bufs_and_sems[NBUF]
    gsem = bufs_and_sems[NBUF + 1:2 * NBUF + 1]
    ssem = bufs_and_sems[2 * NBUF + 1:]

    pltpu.sync_copy(
        gh.at[pl.ds(sid * ROWS_PER_SUB, ROWS_PER_SUB)],
        acc.at[pl.ds(sid * ROWS_PER_SUB, ROWS_PER_SUB)],
    )
    pltpu.sync_copy(src_hbm.at[sid], srcv)
    pltpu.sync_copy(dst_hbm.at[sid], dstv)
    plsc.subcore_barrier()

    for i in range(NBUF):
        pltpu.async_copy(gh.at[srcv.at[i]], bufs[i], gsem[i])

    @pl.loop(0, NCHUNK2 // NBUF - 1)
    def _(t):
        c0 = t * NBUF
        for i in range(NBUF):
            pltpu.make_async_copy(gh.at[srcv.at[c0 + i]], bufs[i],
                                  gsem[i]).wait()
            pltpu.async_copy(bufs[i], acc.at[dstv.at[c0 + i]], ssem[i],
                             add=True)
        for i in range(NBUF):
            pltpu.make_async_copy(bufs[i], acc.at[dstv.at[c0 + i]],
                                  ssem[i]).wait()
            pltpu.async_copy(gh.at[srcv.at[c0 + NBUF + i]], bufs[i], gsem[i])

    cl = NCHUNK2 - NBUF
    for i in range(NBUF):
        pltpu.make_async_copy(gh.at[srcv.at[cl + i]], bufs[i], gsem[i]).wait()
        pltpu.async_copy(bufs[i], acc.at[dstv.at[cl + i]], ssem[i], add=True)
    for i in range(NBUF):
        pltpu.make_async_copy(bufs[i], acc.at[dstv.at[cl + i]],
                              ssem[i]).wait()

    plsc.subcore_barrier()
    pltpu.sync_copy(
        acc.at[pl.ds(sid * ROWS_PER_SUB, ROWS_PER_SUB)],
        out_hbm.at[cid].at[pl.ds(sid * ROWS_PER_SUB, ROWS_PER_SUB)],
    )


def _agg_call(g2h, src_p, dst_p):
    k = pl.kernel(
        _agg_body,
        out_type=jax.ShapeDtypeStruct((2, NPAD, HD), jnp.float32),
        mesh=_vector_mesh(),
        compiler_params=_SC_PARAMS,
        scratch_types=[
            pltpu.VMEM((NCHUNK2, CH), jnp.int32),
            pltpu.VMEM((NCHUNK2, CH), jnp.int32),
        ] + [pltpu.VMEM((CH, HD), jnp.float32)] * NBUF + [
            pltpu.VMEM_SHARED((NPAD, HD), jnp.float32),
        ] + [pltpu.SemaphoreType.DMA] * (2 * NBUF),
    )
    return k(g2h, src_p, dst_p)


def _tcdeg_body(degp_ref, dinv_ref):
    deg = jnp.sum(degp_ref[...], axis=0) + 1.0
    dinv_ref[...] = lax.rsqrt(deg)


def _tc0_body(xp_ref, w1_ref, h_ref):
    h_ref[...] = jnp.dot(xp_ref[...], w1_ref[...], precision=_HIGH,
                         preferred_element_type=jnp.float32)


def _store_halves(out_ref, y):
    out_ref[0] = y[:, :HD]
    out_ref[1] = y[:, HD:]


def _tc1_body(h_ref, dinv_ref, g1_ref):
    _store_halves(g1_ref, h_ref[...] * dinv_ref[...])


def _tc2_body(dinv_ref, p_ref, b_ref, w_ref, out_ref):
    dinv = dinv_ref[...]
    u = jnp.concatenate([p_ref[0], p_ref[1]], axis=1)
    z = jnp.maximum(u * dinv + b_ref[...], 0.0)
    _store_halves(out_ref, jnp.dot(z, w_ref[...], precision=_HIGH,
                                   preferred_element_type=jnp.float32) * dinv)


def _tc3_body(dinv_ref, p_ref, b_ref, batch_ref, wl_ref, bl_ref,
              out_ref):
    dinv = dinv_ref[...]
    u = jnp.concatenate([p_ref[0], p_ref[1]], axis=1)
    z = jnp.maximum(u * dinv + b_ref[...], 0.0)
    gid = lax.broadcasted_iota(jnp.int32, (1, B), 1)
    m = (batch_ref[...] == gid).astype(jnp.float32)
    sums = lax.dot_general(m, z, (((0,), (0,)), ((), ())), precision=_HIGH,
                           preferred_element_type=jnp.float32)
    counts = jnp.sum(m, axis=0)[:, None]
    pooled = sums / jnp.maximum(counts, 1.0)
    out_ref[...] = jnp.dot(pooled, wl_ref[...], precision=_HIGH,
                           preferred_element_type=jnp.float32) + bl_ref[...]


def _tc_call(body, out_shape, *args):
    return pl.pallas_call(body, out_shape=out_shape)(*args)


def kernel(x, ei, batch, W1, b1, W2, b2, Wl, bl):
    pad = EPAD - E
    idx = jnp.arange(pad, dtype=jnp.int32)
    src_flat = jnp.concatenate([ei[0], (idx * 37) % N])
    dst_flat = jnp.concatenate([ei[1], N + idx % (NPAD - N)])
    src_p = src_flat.reshape(16, NCHUNK2, CH)
    dst_p = dst_flat.reshape(16, NCHUNK2, CH)
    dst_deg = ei[1].reshape(NW, NCHD, CHD)
    x_p = jnp.pad(x, ((0, NPAD - N), (0, 0)))
    batch_p = jnp.pad(batch, (0, NPAD - N), constant_values=B).reshape(
        NPAD, 1)

    degp = _deg_call(dst_deg)
    dinv = _tc_call(_tcdeg_body,
                    jax.ShapeDtypeStruct((CHD, D), jnp.float32),
                    degp).reshape(NPAD, 1)

    h1 = _tc_call(_tc0_body, jax.ShapeDtypeStruct((NPAD, D), jnp.float32),
                  x_p, W1)
    g1 = _tc_call(_tc1_body, jax.ShapeDtypeStruct((2, NPAD, HD), jnp.float32),
                  h1, dinv)
    p1 = _agg_call(g1, src_p, dst_p)
    g2 = _tc_call(_tc2_body, jax.ShapeDtypeStruct((2, NPAD, HD), jnp.float32),
                  dinv, p1, b1.reshape(1, D), W2)
    p2 = _agg_call(g2, src_p, dst_p)
    out = _tc_call(_tc3_body, jax.ShapeDtypeStruct((B, D), jnp.float32),
                   dinv, p2, b2.reshape(1, D), batch_p, Wl,
                   bl.reshape(1, D))
    return out
